# baseline (speedup 1.0000x reference)
import functools

import jax
import jax.numpy as jnp
from jax import lax
from jax.experimental import pallas as pl
from jax.experimental.pallas import tpu as pltpu
from jax.experimental.pallas import tpu_sc as plsc

N_NODES = 10240
N_EDGES = 327680
F_IN = 128
C = 16

NC = 2
NS = 16
NW = NC * NS
EPT = N_EDGES // NW
CHUNK = 128
NCHUNK = EPT // CHUNK
ROWS_PT = N_NODES // NS
NBUF = 16

BN_SCALE = 1.0 / (1.0 + 1e-5) ** 0.5


def _sc_segment_sum(p, src, dst):
  mesh = plsc.VectorSubcoreMesh(core_axis_name="c", subcore_axis_name="s")

  @functools.partial(
      pl.kernel,
      mesh=mesh,
      compiler_params=pltpu.CompilerParams(use_tc_tiling_on_sc=False),
      out_type=jax.ShapeDtypeStruct((NC, N_NODES, C), jnp.float32),
      scratch_types=[
          pltpu.VMEM((NCHUNK, CHUNK), jnp.int32),
          pltpu.VMEM((NCHUNK, CHUNK), jnp.int32),
          pltpu.VMEM((NBUF, CHUNK, C), jnp.float32),
          pltpu.VMEM((ROWS_PT, C), jnp.float32),
          pltpu.VMEM_SHARED((N_NODES, C), jnp.float32),
          pltpu.SemaphoreType.DMA((NBUF,)),
      ],
  )
  def k(p_hbm, src_hbm, dst_hbm, out_hbm,
        src_v, dst_v, bufs, zbuf, acc_sh, gsem):
    cid = lax.axis_index("c")
    sid = lax.axis_index("s")
    wid = sid * NC + cid

    def zrow(i, carry):
      zbuf[i, :] = jnp.zeros((C,), jnp.float32)
      return carry
    lax.fori_loop(0, ROWS_PT, zrow, 0)
    pltpu.sync_copy(zbuf, acc_sh.at[pl.ds(sid * ROWS_PT, ROWS_PT)])

    pltpu.sync_copy(src_hbm.at[wid], src_v)
    pltpu.sync_copy(dst_hbm.at[wid], dst_v)

    for b in range(NBUF):
      pltpu.async_copy(p_hbm.at[src_v.at[b]], bufs.at[b], gsem.at[b])
    plsc.subcore_barrier()

    def body(i, carry):
      for b in range(NBUF):
        ch = i * NBUF + b
        pltpu.make_async_copy(p_hbm.at[src_v.at[ch]], bufs.at[b],
                              gsem.at[b]).wait()
        pltpu.sync_copy(bufs.at[b], acc_sh.at[dst_v.at[ch]], add=True)
        nxt = ch + NBUF

        @pl.when(nxt < NCHUNK)
        def _():
          pltpu.async_copy(p_hbm.at[src_v.at[nxt]], bufs.at[b], gsem.at[b])
      return carry
    lax.fori_loop(0, NCHUNK // NBUF, body, 0, unroll=False)
    plsc.subcore_barrier()

    pltpu.sync_copy(acc_sh.at[pl.ds(sid * ROWS_PT, ROWS_PT)], zbuf)
    pltpu.sync_copy(zbuf, out_hbm.at[cid, pl.ds(sid * ROWS_PT, ROWS_PT)])

  return k(p, src, dst)


def _tc_project_in(x, w_rel, w_root):
  blk = 2048

  def body(x_ref, wrel_ref, wroot_ref, p_ref, r_ref):
    xb = x_ref[...]
    p_ref[...] = jnp.dot(xb, wrel_ref[...], preferred_element_type=jnp.float32)
    r_ref[...] = jnp.dot(xb, wroot_ref[...], preferred_element_type=jnp.float32)

  return pl.pallas_call(
      body,
      grid=(N_NODES // blk,),
      in_specs=[
          pl.BlockSpec((blk, F_IN), lambda i: (i, 0)),
          pl.BlockSpec((F_IN, C), lambda i: (0, 0)),
          pl.BlockSpec((F_IN, C), lambda i: (0, 0)),
      ],
      out_specs=[
          pl.BlockSpec((blk, C), lambda i: (i, 0)),
          pl.BlockSpec((blk, C), lambda i: (i, 0)),
      ],
      out_shape=[
          jax.ShapeDtypeStruct((N_NODES, C), jnp.float32),
          jax.ShapeDtypeStruct((N_NODES, C), jnp.float32),
      ],
  )(x, w_rel, w_root)


def _tc_combine_project(agg, r, b_rel, bn_g, bn_b, wn_rel, wn_root):
  blk = 2048

  def body(agg_ref, r_ref, b_ref, g_ref, bb_ref, wrel_ref, wroot_ref,
           p_ref, rn_ref):
    conv = agg_ref[0] + agg_ref[1] + r_ref[...] + b_ref[...]
    h = jnp.maximum(conv, 0.0) * (g_ref[...] * BN_SCALE) + bb_ref[...]
    p_ref[...] = jnp.dot(h, wrel_ref[...], preferred_element_type=jnp.float32)
    rn_ref[...] = jnp.dot(h, wroot_ref[...], preferred_element_type=jnp.float32)

  return pl.pallas_call(
      body,
      grid=(N_NODES // blk,),
      in_specs=[
          pl.BlockSpec((NC, blk, C), lambda i: (0, i, 0)),
          pl.BlockSpec((blk, C), lambda i: (i, 0)),
          pl.BlockSpec((1, C), lambda i: (0, 0)),
          pl.BlockSpec((1, C), lambda i: (0, 0)),
          pl.BlockSpec((1, C), lambda i: (0, 0)),
          pl.BlockSpec((C, C), lambda i: (0, 0)),
          pl.BlockSpec((C, C), lambda i: (0, 0)),
      ],
      out_specs=[
          pl.BlockSpec((blk, C), lambda i: (i, 0)),
          pl.BlockSpec((blk, C), lambda i: (i, 0)),
      ],
      out_shape=[
          jax.ShapeDtypeStruct((N_NODES, C), jnp.float32),
          jax.ShapeDtypeStruct((N_NODES, C), jnp.float32),
      ],
  )(agg, r, b_rel, bn_g, bn_b, wn_rel, wn_root)


def _tc_finish(agg, r, b_rel, bn_g, bn_b, lin1_w, lin1_b, lin2_w, lin2_b):
  groups = N_NODES // F_IN

  def body(agg_ref, r_ref, b_ref, g_ref, bb_ref,
           w1_ref, b1_ref, w2_ref, b2_ref, out_ref):
    conv = agg_ref[0] + agg_ref[1] + r_ref[...] + b_ref[...]
    h = jnp.maximum(conv, 0.0) * (g_ref[...] * BN_SCALE) + bb_ref[...]
    hm = jnp.mean(h.reshape(groups, F_IN, C), axis=1)
    h2 = jnp.maximum(
        jnp.dot(hm, w1_ref[...], preferred_element_type=jnp.float32)
        + b1_ref[...], 0.0)
    out_ref[...] = (
        jnp.dot(h2, w2_ref[...], preferred_element_type=jnp.float32)
        + b2_ref[...])

  out = pl.pallas_call(
      body,
      out_shape=jax.ShapeDtypeStruct((groups, 1), jnp.float32),
  )(agg, r, b_rel, bn_g, bn_b, lin1_w, lin1_b, lin2_w, lin2_b)
  return out[:, 0]


def kernel(x, edge_index, W1_rel, b1_rel, W1_root, W2_rel, b2_rel, W2_root,
           W3_rel, b3_rel, W3_root, bn1_g, bn1_b, bn2_g, bn2_b, bn3_g, bn3_b,
           lin1_W, lin1_b, lin2_W, lin2_b):
  src = edge_index[0].reshape(NW, NCHUNK, CHUNK)
  dst = edge_index[1].reshape(NW, NCHUNK, CHUNK)

  p1, r1 = _tc_project_in(x, W1_rel, W1_root)
  agg1 = _sc_segment_sum(p1, src, dst)
  p2, r2 = _tc_combine_project(
      agg1, r1, b1_rel.reshape(1, C), bn1_g.reshape(1, C), bn1_b.reshape(1, C),
      W2_rel, W2_root)
  agg2 = _sc_segment_sum(p2, src, dst)
  p3, r3 = _tc_combine_project(
      agg2, r2, b2_rel.reshape(1, C), bn2_g.reshape(1, C), bn2_b.reshape(1, C),
      W3_rel, W3_root)
  agg3 = _sc_segment_sum(p3, src, dst)
  return _tc_finish(
      agg3, r3, b3_rel.reshape(1, C), bn3_g.reshape(1, C), bn3_b.reshape(1, C),
      lin1_W, lin1_b.reshape(1, C), lin2_W, lin2_b.reshape(1, 1))


# speedup vs baseline: 1.3958x; 1.3958x over previous
import functools

import jax
import jax.numpy as jnp
from jax import lax
from jax.experimental import pallas as pl
from jax.experimental.pallas import tpu as pltpu
from jax.experimental.pallas import tpu_sc as plsc

N_NODES = 10240
N_EDGES = 327680
F_IN = 128
C = 16
PACK = 128 // C
NP = N_NODES // PACK

NC = 2
NS = 16
NW = NC * NS
EPT = N_EDGES // NW
CHUNK = 128
NCHUNK = EPT // CHUNK
ROWS_PT = N_NODES // NS
NBUF = 16

BN_SCALE = 1.0 / (1.0 + 1e-5) ** 0.5


def _sc_segment_sum(p, src, dst):
  mesh = plsc.VectorSubcoreMesh(core_axis_name="c", subcore_axis_name="s")

  @functools.partial(
      pl.kernel,
      mesh=mesh,
      compiler_params=pltpu.CompilerParams(use_tc_tiling_on_sc=False),
      out_type=jax.ShapeDtypeStruct((NC, N_NODES, C), jnp.float32),
      scratch_types=[
          pltpu.VMEM((NCHUNK, CHUNK), jnp.int32),
          pltpu.VMEM((NCHUNK, CHUNK), jnp.int32),
          pltpu.VMEM((NBUF, CHUNK, C), jnp.float32),
          pltpu.VMEM((ROWS_PT, C), jnp.float32),
          pltpu.VMEM_SHARED((N_NODES, C), jnp.float32),
          pltpu.SemaphoreType.DMA((NBUF,)),
      ],
  )
  def k(p_hbm, src_hbm, dst_hbm, out_hbm,
        src_v, dst_v, bufs, zbuf, acc_sh, gsem):
    cid = lax.axis_index("c")
    sid = lax.axis_index("s")
    wid = sid * NC + cid

    def zrow(i, carry):
      zbuf[i, :] = jnp.zeros((C,), jnp.float32)
      return carry
    lax.fori_loop(0, ROWS_PT, zrow, 0)
    pltpu.sync_copy(zbuf, acc_sh.at[pl.ds(sid * ROWS_PT, ROWS_PT)])

    pltpu.sync_copy(src_hbm.at[wid], src_v)
    pltpu.sync_copy(dst_hbm.at[wid], dst_v)

    for b in range(NBUF):
      pltpu.async_copy(p_hbm.at[src_v.at[b]], bufs.at[b], gsem.at[b])
    plsc.subcore_barrier()

    def body(i, carry):
      for b in range(NBUF):
        ch = i * NBUF + b
        pltpu.make_async_copy(p_hbm.at[src_v.at[ch]], bufs.at[b],
                              gsem.at[b]).wait()
        pltpu.sync_copy(bufs.at[b], acc_sh.at[dst_v.at[ch]], add=True)
        nxt = ch + NBUF

        @pl.when(nxt < NCHUNK)
        def _():
          pltpu.async_copy(p_hbm.at[src_v.at[nxt]], bufs.at[b], gsem.at[b])
      return carry
    lax.fori_loop(0, NCHUNK // NBUF, body, 0, unroll=False)
    plsc.subcore_barrier()

    pltpu.sync_copy(acc_sh.at[pl.ds(sid * ROWS_PT, ROWS_PT)], zbuf)
    pltpu.sync_copy(zbuf, out_hbm.at[cid, pl.ds(sid * ROWS_PT, ROWS_PT)])

  return k(p, src, dst)


def _tc_project_in(xr, wb_rel, wb_root):
  def body(x_ref, wrel_ref, wroot_ref, p_ref, r_ref):
    xb = x_ref[...]
    p_ref[...] = jnp.dot(xb, wrel_ref[...], preferred_element_type=jnp.float32)
    r_ref[...] = jnp.dot(xb, wroot_ref[...], preferred_element_type=jnp.float32)

  return pl.pallas_call(
      body,
      out_shape=[
          jax.ShapeDtypeStruct((NP, 128), jnp.float32),
          jax.ShapeDtypeStruct((NP, 128), jnp.float32),
      ],
  )(xr, wb_rel, wb_root)


def _tc_combine_project(agg, r, bvec, svec, bbvec, wd_rel, wd_root):
  def body(agg_ref, r_ref, b_ref, s_ref, bb_ref, wrel_ref, wroot_ref,
           p_ref, rn_ref):
    conv = agg_ref[0] + agg_ref[1] + r_ref[...] + b_ref[...]
    h = jnp.maximum(conv, 0.0) * s_ref[...] + bb_ref[...]
    p_ref[...] = jnp.dot(h, wrel_ref[...], preferred_element_type=jnp.float32)
    rn_ref[...] = jnp.dot(h, wroot_ref[...], preferred_element_type=jnp.float32)

  return pl.pallas_call(
      body,
      out_shape=[
          jax.ShapeDtypeStruct((NP, 128), jnp.float32),
          jax.ShapeDtypeStruct((NP, 128), jnp.float32),
      ],
  )(agg, r, bvec, svec, bbvec, wd_rel, wd_root)


def _tc_finish(agg, r, bvec, svec, bbvec, lin1_w, lin1_b, lin2_w, lin2_b):
  groups = N_NODES // F_IN
  rows_per_group = F_IN // PACK

  def body(agg_ref, r_ref, b_ref, s_ref, bb_ref,
           w1_ref, b1_ref, w2_ref, b2_ref, out_ref):
    conv = agg_ref[0] + agg_ref[1] + r_ref[...] + b_ref[...]
    h = jnp.maximum(conv, 0.0) * s_ref[...] + bb_ref[...]
    row_g = lax.broadcasted_iota(jnp.int32, (groups, NP), 1) // rows_per_group
    gsel = (row_g == lax.broadcasted_iota(jnp.int32, (groups, NP), 0)
            ).astype(jnp.float32)
    m1 = jnp.dot(gsel, h, preferred_element_type=jnp.float32)
    fold = (lax.broadcasted_iota(jnp.int32, (128, C), 0) % C ==
            lax.broadcasted_iota(jnp.int32, (128, C), 1)).astype(jnp.float32)
    hm = jnp.dot(m1, fold, preferred_element_type=jnp.float32) * (1.0 / F_IN)
    h2 = jnp.maximum(
        jnp.dot(hm, w1_ref[...], preferred_element_type=jnp.float32)
        + b1_ref[...], 0.0)
    out_ref[...] = (
        jnp.dot(h2, w2_ref[...], preferred_element_type=jnp.float32)
        + b2_ref[...])

  out = pl.pallas_call(
      body,
      out_shape=jax.ShapeDtypeStruct((groups, 1), jnp.float32),
  )(agg, r, bvec, svec, bbvec, lin1_w, lin1_b, lin2_w, lin2_b)
  return out[:, 0]


def _expand_in_weight(w):
  a = jnp.arange(PACK)
  big = jnp.zeros((PACK, F_IN, PACK, C), jnp.float32)
  big = big.at[a, :, a, :].set(jnp.broadcast_to(w, (PACK, F_IN, C)))
  return big.reshape(PACK * F_IN, PACK * C)


def _blockdiag8(w):
  a = jnp.arange(PACK)
  big = jnp.zeros((PACK, C, PACK, C), jnp.float32)
  big = big.at[a, :, a, :].set(jnp.broadcast_to(w, (PACK, C, C)))
  return big.reshape(PACK * C, PACK * C)


def _tile8(v):
  return jnp.tile(v, (PACK,)).reshape(1, PACK * C)


def kernel(x, edge_index, W1_rel, b1_rel, W1_root, W2_rel, b2_rel, W2_root,
           W3_rel, b3_rel, W3_root, bn1_g, bn1_b, bn2_g, bn2_b, bn3_g, bn3_b,
           lin1_W, lin1_b, lin2_W, lin2_b):
  src = edge_index[0].reshape(NW, NCHUNK, CHUNK)
  dst = edge_index[1].reshape(NW, NCHUNK, CHUNK)

  xr = x.reshape(NP, PACK * F_IN)
  b1v, s1v, bb1v = _tile8(b1_rel), _tile8(bn1_g * BN_SCALE), _tile8(bn1_b)
  b2v, s2v, bb2v = _tile8(b2_rel), _tile8(bn2_g * BN_SCALE), _tile8(bn2_b)
  b3v, s3v, bb3v = _tile8(b3_rel), _tile8(bn3_g * BN_SCALE), _tile8(bn3_b)

  p1, r1 = _tc_project_in(xr, _expand_in_weight(W1_rel),
                          _expand_in_weight(W1_root))
  agg1 = _sc_segment_sum(p1.reshape(N_NODES, C), src, dst)
  p2, r2 = _tc_combine_project(agg1.reshape(NC, NP, PACK * C), r1,
                               b1v, s1v, bb1v,
                               _blockdiag8(W2_rel), _blockdiag8(W2_root))
  agg2 = _sc_segment_sum(p2.reshape(N_NODES, C), src, dst)
  p3, r3 = _tc_combine_project(agg2.reshape(NC, NP, PACK * C), r2,
                               b2v, s2v, bb2v,
                               _blockdiag8(W3_rel), _blockdiag8(W3_root))
  agg3 = _sc_segment_sum(p3.reshape(N_NODES, C), src, dst)
  return _tc_finish(agg3.reshape(NC, NP, PACK * C), r3, b3v, s3v, bb3v,
                    lin1_W, lin1_b.reshape(1, C), lin2_W, lin2_b.reshape(1, 1))


# speedup vs baseline: 1.4964x; 1.0721x over previous
import functools

import jax
import jax.numpy as jnp
from jax import lax
from jax.experimental import pallas as pl
from jax.experimental.pallas import tpu as pltpu
from jax.experimental.pallas import tpu_sc as plsc

N_NODES = 10240
N_EDGES = 327680
F_IN = 128
C = 16
PACK = 128 // C
NP = N_NODES // PACK

NC = 2
NS = 16
NW = NC * NS
EPT = N_EDGES // NW
CHUNK = 128
NCHUNK = EPT // CHUNK
ROWS_PT = N_NODES // NS
NBUF = 16

BN_SCALE = 1.0 / (1.0 + 1e-5) ** 0.5


def _sc_segment_sum(p, src, dst, zrows):
  mesh = plsc.VectorSubcoreMesh(core_axis_name="c", subcore_axis_name="s")

  @functools.partial(
      pl.kernel,
      mesh=mesh,
      compiler_params=pltpu.CompilerParams(use_tc_tiling_on_sc=False),
      out_type=jax.ShapeDtypeStruct((NC, N_NODES, C), jnp.float32),
      scratch_types=[
          pltpu.VMEM((NCHUNK, CHUNK), jnp.int32),
          pltpu.VMEM((NCHUNK, CHUNK), jnp.int32),
          pltpu.VMEM((NBUF, CHUNK, C), jnp.float32),
          pltpu.VMEM((ROWS_PT, C), jnp.float32),
          pltpu.VMEM_SHARED((N_NODES, C), jnp.float32),
          pltpu.SemaphoreType.DMA((NBUF,)),
      ],
  )
  def k(p_hbm, src_hbm, dst_hbm, z_hbm, out_hbm,
        src_v, dst_v, bufs, zbuf, acc_sh, gsem):
    cid = lax.axis_index("c")
    sid = lax.axis_index("s")
    wid = sid * NC + cid

    pltpu.sync_copy(src_hbm.at[wid], src_v)
    pltpu.sync_copy(dst_hbm.at[wid], dst_v)

    for b in range(NBUF):
      pltpu.async_copy(p_hbm.at[src_v.at[b]], bufs.at[b], gsem.at[b])

    pltpu.sync_copy(z_hbm, acc_sh.at[pl.ds(sid * ROWS_PT, ROWS_PT)])
    plsc.subcore_barrier()

    def body(i, carry):
      for b in range(NBUF):
        ch = i * NBUF + b
        pltpu.make_async_copy(p_hbm.at[src_v.at[ch]], bufs.at[b],
                              gsem.at[b]).wait()
        pltpu.sync_copy(bufs.at[b], acc_sh.at[dst_v.at[ch]], add=True)
        nxt = ch + NBUF

        @pl.when(nxt < NCHUNK)
        def _():
          pltpu.async_copy(p_hbm.at[src_v.at[nxt]], bufs.at[b], gsem.at[b])
      return carry
    lax.fori_loop(0, NCHUNK // NBUF, body, 0, unroll=False)
    plsc.subcore_barrier()

    pltpu.sync_copy(acc_sh.at[pl.ds(sid * ROWS_PT, ROWS_PT)], zbuf)
    pltpu.sync_copy(zbuf, out_hbm.at[cid, pl.ds(sid * ROWS_PT, ROWS_PT)])

  return k(p, src, dst, zrows)


def _tc_project_in(xr, wb_rel, wb_root, edge_index):
  def body(x_ref, wrel_ref, wroot_ref, e_ref, p_ref, r_ref, e_out_ref):
    xb = x_ref[...]
    p_ref[...] = jnp.dot(xb, wrel_ref[...], preferred_element_type=jnp.float32)
    r_ref[...] = jnp.dot(xb, wroot_ref[...], preferred_element_type=jnp.float32)
    e_out_ref[...] = e_ref[...].reshape(2, N_EDGES // CHUNK, CHUNK)

  return pl.pallas_call(
      body,
      out_shape=[
          jax.ShapeDtypeStruct((NP, 128), jnp.float32),
          jax.ShapeDtypeStruct((NP, 128), jnp.float32),
          jax.ShapeDtypeStruct((2, N_EDGES // CHUNK, CHUNK), jnp.int32),
      ],
  )(xr, wb_rel, wb_root, edge_index)


def _tc_combine_project(agg, r, bvec, svec, bbvec, wd_rel, wd_root):
  def body(agg_ref, r_ref, b_ref, s_ref, bb_ref, wrel_ref, wroot_ref,
           p_ref, rn_ref):
    conv = agg_ref[0] + agg_ref[1] + r_ref[...] + b_ref[...]
    h = jnp.maximum(conv, 0.0) * s_ref[...] + bb_ref[...]
    p_ref[...] = jnp.dot(h, wrel_ref[...], preferred_element_type=jnp.float32)
    rn_ref[...] = jnp.dot(h, wroot_ref[...], preferred_element_type=jnp.float32)

  return pl.pallas_call(
      body,
      out_shape=[
          jax.ShapeDtypeStruct((NP, 128), jnp.float32),
          jax.ShapeDtypeStruct((NP, 128), jnp.float32),
      ],
  )(agg, r, bvec, svec, bbvec, wd_rel, wd_root)


def _tc_finish(agg, r, bvec, svec, bbvec, lin1_w, lin1_b, lin2_w, lin2_b):
  groups = N_NODES // F_IN
  rows_per_group = F_IN // PACK

  def body(agg_ref, r_ref, b_ref, s_ref, bb_ref,
           w1_ref, b1_ref, w2_ref, b2_ref, out_ref):
    conv = agg_ref[0] + agg_ref[1] + r_ref[...] + b_ref[...]
    h = jnp.maximum(conv, 0.0) * s_ref[...] + bb_ref[...]
    row_g = lax.broadcasted_iota(jnp.int32, (groups, NP), 1) // rows_per_group
    gsel = (row_g == lax.broadcasted_iota(jnp.int32, (groups, NP), 0)
            ).astype(jnp.float32)
    m1 = jnp.dot(gsel, h, preferred_element_type=jnp.float32)
    fold = (lax.broadcasted_iota(jnp.int32, (128, C), 0) % C ==
            lax.broadcasted_iota(jnp.int32, (128, C), 1)).astype(jnp.float32)
    hm = jnp.dot(m1, fold, preferred_element_type=jnp.float32) * (1.0 / F_IN)
    h2 = jnp.maximum(
        jnp.dot(hm, w1_ref[...], preferred_element_type=jnp.float32)
        + b1_ref[...], 0.0)
    out_ref[...] = (
        jnp.dot(h2, w2_ref[...], preferred_element_type=jnp.float32)
        + b2_ref[...])

  out = pl.pallas_call(
      body,
      out_shape=jax.ShapeDtypeStruct((groups, 1), jnp.float32),
  )(agg, r, bvec, svec, bbvec, lin1_w, lin1_b, lin2_w, lin2_b)
  return out[:, 0]


def _expand_in_weight(w):
  a = jnp.arange(PACK)
  big = jnp.zeros((PACK, F_IN, PACK, C), jnp.float32)
  big = big.at[a, :, a, :].set(jnp.broadcast_to(w, (PACK, F_IN, C)))
  return big.reshape(PACK * F_IN, PACK * C)


def _blockdiag8(w):
  a = jnp.arange(PACK)
  big = jnp.zeros((PACK, C, PACK, C), jnp.float32)
  big = big.at[a, :, a, :].set(jnp.broadcast_to(w, (PACK, C, C)))
  return big.reshape(PACK * C, PACK * C)


def _tile8(v):
  return jnp.tile(v, (PACK,)).reshape(1, PACK * C)


def kernel(x, edge_index, W1_rel, b1_rel, W1_root, W2_rel, b2_rel, W2_root,
           W3_rel, b3_rel, W3_root, bn1_g, bn1_b, bn2_g, bn2_b, bn3_g, bn3_b,
           lin1_W, lin1_b, lin2_W, lin2_b):
  zrows = jnp.zeros((ROWS_PT, C), jnp.float32)

  xr = x.reshape(NP, PACK * F_IN)
  b1v, s1v, bb1v = _tile8(b1_rel), _tile8(bn1_g * BN_SCALE), _tile8(bn1_b)
  b2v, s2v, bb2v = _tile8(b2_rel), _tile8(bn2_g * BN_SCALE), _tile8(bn2_b)
  b3v, s3v, bb3v = _tile8(b3_rel), _tile8(bn3_g * BN_SCALE), _tile8(bn3_b)

  p1, r1, edges = _tc_project_in(xr, _expand_in_weight(W1_rel),
                                 _expand_in_weight(W1_root), edge_index)
  src = edges[0].reshape(NW, NCHUNK, CHUNK)
  dst = edges[1].reshape(NW, NCHUNK, CHUNK)
  agg1 = _sc_segment_sum(p1.reshape(N_NODES, C), src, dst, zrows)
  p2, r2 = _tc_combine_project(agg1.reshape(NC, NP, PACK * C), r1,
                               b1v, s1v, bb1v,
                               _blockdiag8(W2_rel), _blockdiag8(W2_root))
  agg2 = _sc_segment_sum(p2.reshape(N_NODES, C), src, dst, zrows)
  p3, r3 = _tc_combine_project(agg2.reshape(NC, NP, PACK * C), r2,
                               b2v, s2v, bb2v,
                               _blockdiag8(W3_rel), _blockdiag8(W3_root))
  agg3 = _sc_segment_sum(p3.reshape(N_NODES, C), src, dst, zrows)
  return _tc_finish(agg3.reshape(NC, NP, PACK * C), r3, b3v, s3v, bb3v,
                    lin1_W, lin1_b.reshape(1, C), lin2_W, lin2_b.reshape(1, 1))


# speedup vs baseline: 1.6493x; 1.1021x over previous
import functools

import jax
import jax.numpy as jnp
from jax import lax
from jax.experimental import pallas as pl
from jax.experimental.pallas import tpu as pltpu
from jax.experimental.pallas import tpu_sc as plsc

N_NODES = 10240
N_EDGES = 327680
F_IN = 128
C = 16
PACK = 128 // C
NP = N_NODES // PACK

NC = 2
NS = 16
NW = NC * NS
EPT = N_EDGES // NW
CHUNK = 128
NCHUNK = EPT // CHUNK
ROWS_PT = N_NODES // NS
NBUF = 16

BN_SCALE = 1.0 / (1.0 + 1e-5) ** 0.5


def _sc_segment_sum(p, src, dst, zrows):
  mesh = plsc.VectorSubcoreMesh(core_axis_name="c", subcore_axis_name="s")

  @functools.partial(
      pl.kernel,
      mesh=mesh,
      compiler_params=pltpu.CompilerParams(use_tc_tiling_on_sc=False),
      out_type=jax.ShapeDtypeStruct((NC, N_NODES, C), jnp.float32),
      scratch_types=[
          pltpu.VMEM((NCHUNK, CHUNK), jnp.int32),
          pltpu.VMEM((NCHUNK, CHUNK), jnp.int32),
          pltpu.VMEM((NBUF, CHUNK, C), jnp.float32),
          pltpu.VMEM((ROWS_PT, C), jnp.float32),
          pltpu.VMEM_SHARED((N_NODES, C), jnp.float32),
          pltpu.SemaphoreType.DMA((NBUF,)),
      ],
  )
  def k(p_hbm, src_hbm, dst_hbm, z_hbm, out_hbm,
        src_v, dst_v, bufs, zbuf, acc_sh, gsem):
    cid = lax.axis_index("c")
    sid = lax.axis_index("s")
    wid = sid * NC + cid

    pltpu.sync_copy(src_hbm.at[wid], src_v)
    pltpu.sync_copy(dst_hbm.at[wid], dst_v)

    for b in range(NBUF):
      pltpu.async_copy(p_hbm.at[src_v.at[b]], bufs.at[b], gsem.at[b])

    pltpu.sync_copy(z_hbm, acc_sh.at[pl.ds(sid * ROWS_PT, ROWS_PT)])
    plsc.subcore_barrier()

    def body(i, carry):
      for b in range(NBUF):
        ch = i * NBUF + b
        pltpu.make_async_copy(p_hbm.at[src_v.at[ch]], bufs.at[b],
                              gsem.at[b]).wait()
        pltpu.sync_copy(bufs.at[b], acc_sh.at[dst_v.at[ch]], add=True)
        nxt = ch + NBUF

        @pl.when(nxt < NCHUNK)
        def _():
          pltpu.async_copy(p_hbm.at[src_v.at[nxt]], bufs.at[b], gsem.at[b])
      return carry
    lax.fori_loop(0, NCHUNK // NBUF, body, 0, unroll=False)
    plsc.subcore_barrier()

    pltpu.sync_copy(acc_sh.at[pl.ds(sid * ROWS_PT, ROWS_PT)], zbuf)
    pltpu.sync_copy(zbuf, out_hbm.at[cid, pl.ds(sid * ROWS_PT, ROWS_PT)])

  return k(p, src, dst, zrows)


def _tc_project_in(x, wb_rel, wb_root, edge_index):
  def body(x_ref, wrel_ref, wroot_ref, e_ref, p_ref, r_ref, e_out_ref):
    xb = x_ref[...].reshape(NP, PACK * F_IN)
    p_ref[...] = jnp.dot(xb, wrel_ref[...], preferred_element_type=jnp.float32)
    r_ref[...] = jnp.dot(xb, wroot_ref[...], preferred_element_type=jnp.float32)
    e_out_ref[...] = e_ref[...].reshape(2, N_EDGES // CHUNK, CHUNK)

  return pl.pallas_call(
      body,
      out_shape=[
          jax.ShapeDtypeStruct((NP, 128), jnp.float32),
          jax.ShapeDtypeStruct((NP, 128), jnp.float32),
          jax.ShapeDtypeStruct((2, N_EDGES // CHUNK, CHUNK), jnp.int32),
      ],
  )(x, wb_rel, wb_root, edge_index)


def _tc_combine_project(agg, r, bvec, svec, bbvec, wd_rel, wd_root):
  def body(agg_ref, r_ref, b_ref, s_ref, bb_ref, wrel_ref, wroot_ref,
           p_ref, rn_ref):
    conv = agg_ref[0] + agg_ref[1] + r_ref[...] + b_ref[...]
    h = jnp.maximum(conv, 0.0) * s_ref[...] + bb_ref[...]
    p_ref[...] = jnp.dot(h, wrel_ref[...], preferred_element_type=jnp.float32)
    rn_ref[...] = jnp.dot(h, wroot_ref[...], preferred_element_type=jnp.float32)

  return pl.pallas_call(
      body,
      out_shape=[
          jax.ShapeDtypeStruct((NP, 128), jnp.float32),
          jax.ShapeDtypeStruct((NP, 128), jnp.float32),
      ],
  )(agg, r, bvec, svec, bbvec, wd_rel, wd_root)


def _tc_finish(agg, r, bvec, svec, bbvec, lin1_w, lin1_b, lin2_w, lin2_b):
  groups = N_NODES // F_IN
  rows_per_group = F_IN // PACK

  def body(agg_ref, r_ref, b_ref, s_ref, bb_ref,
           w1_ref, b1_ref, w2_ref, b2_ref, out_ref):
    conv = agg_ref[0] + agg_ref[1] + r_ref[...] + b_ref[...]
    h = jnp.maximum(conv, 0.0) * s_ref[...] + bb_ref[...]
    row_g = lax.broadcasted_iota(jnp.int32, (groups, NP), 1) // rows_per_group
    gsel = (row_g == lax.broadcasted_iota(jnp.int32, (groups, NP), 0)
            ).astype(jnp.float32)
    m1 = jnp.dot(gsel, h, preferred_element_type=jnp.float32)
    fold = (lax.broadcasted_iota(jnp.int32, (128, C), 0) % C ==
            lax.broadcasted_iota(jnp.int32, (128, C), 1)).astype(jnp.float32)
    hm = jnp.dot(m1, fold, preferred_element_type=jnp.float32) * (1.0 / F_IN)
    h2 = jnp.maximum(
        jnp.dot(hm, w1_ref[...], preferred_element_type=jnp.float32)
        + b1_ref[...], 0.0)
    out_ref[...] = (
        jnp.dot(h2, w2_ref[...], preferred_element_type=jnp.float32)
        + b2_ref[...])

  out = pl.pallas_call(
      body,
      out_shape=jax.ShapeDtypeStruct((groups, 1), jnp.float32),
  )(agg, r, bvec, svec, bbvec, lin1_w, lin1_b, lin2_w, lin2_b)
  return out[:, 0]


def _expand_in_weight(w):
  a = jnp.arange(PACK)
  big = jnp.zeros((PACK, F_IN, PACK, C), jnp.float32)
  big = big.at[a, :, a, :].set(jnp.broadcast_to(w, (PACK, F_IN, C)))
  return big.reshape(PACK * F_IN, PACK * C)


def _blockdiag8(w):
  a = jnp.arange(PACK)
  big = jnp.zeros((PACK, C, PACK, C), jnp.float32)
  big = big.at[a, :, a, :].set(jnp.broadcast_to(w, (PACK, C, C)))
  return big.reshape(PACK * C, PACK * C)


def _tile8(v):
  return jnp.tile(v, (PACK,)).reshape(1, PACK * C)


def kernel(x, edge_index, W1_rel, b1_rel, W1_root, W2_rel, b2_rel, W2_root,
           W3_rel, b3_rel, W3_root, bn1_g, bn1_b, bn2_g, bn2_b, bn3_g, bn3_b,
           lin1_W, lin1_b, lin2_W, lin2_b):
  zrows = jnp.zeros((ROWS_PT, C), jnp.float32)

  b1v, s1v, bb1v = _tile8(b1_rel), _tile8(bn1_g * BN_SCALE), _tile8(bn1_b)
  b2v, s2v, bb2v = _tile8(b2_rel), _tile8(bn2_g * BN_SCALE), _tile8(bn2_b)
  b3v, s3v, bb3v = _tile8(b3_rel), _tile8(bn3_g * BN_SCALE), _tile8(bn3_b)

  p1, r1, edges = _tc_project_in(x, _expand_in_weight(W1_rel),
                                 _expand_in_weight(W1_root), edge_index)
  src = edges[0].reshape(NW, NCHUNK, CHUNK)
  dst = edges[1].reshape(NW, NCHUNK, CHUNK)
  agg1 = _sc_segment_sum(p1.reshape(N_NODES, C), src, dst, zrows)
  p2, r2 = _tc_combine_project(agg1.reshape(NC, NP, PACK * C), r1,
                               b1v, s1v, bb1v,
                               _blockdiag8(W2_rel), _blockdiag8(W2_root))
  agg2 = _sc_segment_sum(p2.reshape(N_NODES, C), src, dst, zrows)
  p3, r3 = _tc_combine_project(agg2.reshape(NC, NP, PACK * C), r2,
                               b2v, s2v, bb2v,
                               _blockdiag8(W3_rel), _blockdiag8(W3_root))
  agg3 = _sc_segment_sum(p3.reshape(N_NODES, C), src, dst, zrows)
  return _tc_finish(agg3.reshape(NC, NP, PACK * C), r3, b3v, s3v, bb3v,
                    lin1_W, lin1_b.reshape(1, C), lin2_W, lin2_b.reshape(1, 1))
